# baseline (speedup 1.0000x reference)
import functools

import jax
import jax.numpy as jnp
from jax import lax
from jax.experimental import pallas as pl
from jax.experimental.pallas import tpu as pltpu
from jax.experimental.pallas import tpu_sc as plsc

_VOCAB = (256, 128, 128)
_K = 16
_S = 32
_JOINT = 256 * 128 * 128

_NC = 2
_NS = 16
_NW = _NC * _NS
_L = 16

_ASPLIT = 128
_ASC = 256 - _ASPLIT
_APW = _ASC // _NW


def _tables_body(tm0_ref, tm1_ref, tm2_ref, w0_ref, w1_ref, w2_ref,
                 p0_ref, p1_ref, p2_ref):
    for tm_ref, w_ref, p_ref in ((tm0_ref, w0_ref, p0_ref),
                                 (tm1_ref, w1_ref, p1_ref),
                                 (tm2_ref, w2_ref, p2_ref)):
        x = tm_ref[0]
        w = w_ref[0]
        vals = jnp.sum(x * w, axis=1)
        vals = jnp.abs(vals) + 1e-9
        p_ref[0, 0] = vals / jnp.sum(vals)


def _compute_tables(tm0, tm1, tm2, w0, w1, w2):
    out = pl.pallas_call(
        _tables_body,
        grid=(_K,),
        in_specs=[
            pl.BlockSpec((1, _VOCAB[0], _S * _S), lambda k: (k, 0, 0)),
            pl.BlockSpec((1, _VOCAB[1], _S * _S), lambda k: (k, 0, 0)),
            pl.BlockSpec((1, _VOCAB[2], _S * _S), lambda k: (k, 0, 0)),
            pl.BlockSpec((1, 1, _S * _S), lambda k: (k, 0, 0)),
            pl.BlockSpec((1, 1, _S * _S), lambda k: (k, 0, 0)),
            pl.BlockSpec((1, 1, _S * _S), lambda k: (k, 0, 0)),
        ],
        out_specs=[
            pl.BlockSpec((1, 1, _VOCAB[0]), lambda k: (k, 0, 0)),
            pl.BlockSpec((1, 1, _VOCAB[1]), lambda k: (k, 0, 0)),
            pl.BlockSpec((1, 1, _VOCAB[2]), lambda k: (k, 0, 0)),
        ],
        out_shape=[
            jax.ShapeDtypeStruct((_K, 1, _VOCAB[0]), jnp.float32),
            jax.ShapeDtypeStruct((_K, 1, _VOCAB[1]), jnp.float32),
            jax.ShapeDtypeStruct((_K, 1, _VOCAB[2]), jnp.float32),
        ],
    )(tm0, tm1, tm2, w0, w1, w2)
    return [o.reshape(_K, v) for o, v in zip(out, _VOCAB)]


_P1STRIDE = _K * _VOCAB[1] + 1


def _sc_product_body(p0t_h, p1_h, p2_h, cm0_h, cm1_h, cm2_h,
                     raw_h, psum_h,
                     p0t_v, p1_v, p2_v, p1rep_v, q0_v,
                     cm0_v, cm1_v, cm2_v, ob, sv):
    cid = lax.axis_index("c")
    sid = lax.axis_index("s")
    wid = sid * _NC + cid
    a0 = wid * _APW
    lane = lax.iota(jnp.int32, _L)
    pltpu.sync_copy(p0t_h.at[pl.ds(a0 * _K, _APW * _K)], p0t_v)
    pltpu.sync_copy(p1_h, p1_v)
    pltpu.sync_copy(p2_h, p2_v)
    pltpu.sync_copy(cm0_h, cm0_v)
    pltpu.sync_copy(cm1_h.at[pl.ds(a0, _APW)], cm1_v)
    pltpu.sync_copy(cm2_h.at[pl.ds(a0, _APW)], cm2_v)

    def pre0_body(i, _):
        for cc in range(128 // _L):
            v = cm0_v[i, pl.ds(cc * _L, _L)]
            cm0_v[i, pl.ds(cc * _L, _L)] = v * _L + lane
        return 0

    lax.fori_loop(0, 128, pre0_body, 0)

    lbases = [jnp.full((_L,), l * _P1STRIDE, jnp.int32) + lane
              for l in range(_L)]

    def rep_body(i, _):
        v = p1_v[pl.ds(i * _L, _L)]
        off = i * _L
        for l in range(_L):
            plsc.store_scatter(p1rep_v, [lbases[l] + off], v)
        return 0

    lax.fori_loop(0, (_K * _VOCAB[1]) // _L, rep_body, 0)
    lanestride = lane * _P1STRIDE

    def a_body(a, total):
        a_g = a0 + a
        q0 = p0t_v[pl.ds(a * _K, _K)]
        for k in range(_K):
            q0_v[pl.ds(k * _L, _L)] = jnp.full((_L,), q0[k], jnp.float32)
        idx1 = [cm1_v[a, pl.ds(cc * _L, _L)] * _VOCAB[1] + lanestride
                for cc in range(128 // _L)]

        @plsc.parallel_loop(0, 128 // _L, 1, unroll=2, carry=total)
        def bb_out(bb, tot, a=a, idx1=idx1):
            b0 = bb * _L
            r2v = cm2_v[a, pl.ds(b0, _L)]
            tot = list(tot)
            for j in range(_L):
                b = b0 + j
                r2 = r2v[j]
                b_vec = jnp.full((_L,), b, jnp.int32)
                for cc in range(128 // _L):
                    c0 = cc * _L
                    idx0 = cm0_v[b, pl.ds(c0, _L)]
                    t0 = plsc.load_gather(q0_v, [idx0])
                    t1 = plsc.load_gather(p1rep_v, [idx1[cc] + b_vec])
                    t2 = p2_v[pl.ds(r2 * _VOCAB[2] + c0, _L)]
                    prod = t0 * t1 * t2
                    ob[b, pl.ds(c0, _L)] = prod
                    tot[cc] = tot[cc] + prod
            return tuple(tot)

        total = bb_out
        pltpu.sync_copy(ob, raw_h.at[a_g])
        return total

    zeros = tuple(jnp.zeros((_L,), jnp.float32) for _ in range(128 // _L))
    total = lax.fori_loop(0, _APW, a_body, zeros)
    sv[...] = sum(total[1:], total[0])
    pltpu.sync_copy(sv, psum_h.at[wid])


def _compute_product(p0t, p1, p2, cm0, cm1, cm2):
    mesh = plsc.VectorSubcoreMesh(core_axis_name="c", subcore_axis_name="s",
                                  num_cores=_NC, num_subcores=_NS)
    f = pl.kernel(
        _sc_product_body,
        out_type=[
            jax.ShapeDtypeStruct((_ASC, 128, 128), jnp.float32),
            jax.ShapeDtypeStruct((_NW, _L), jnp.float32),
        ],
        mesh=mesh,
        compiler_params=pltpu.CompilerParams(use_tc_tiling_on_sc=False,
                                             needs_layout_passes=False),
        scratch_types=[
            pltpu.VMEM((_APW * _K,), jnp.float32),
            pltpu.VMEM((_K * _VOCAB[1],), jnp.float32),
            pltpu.VMEM((_K * _VOCAB[2],), jnp.float32),
            pltpu.VMEM((_L * _P1STRIDE,), jnp.float32),
            pltpu.VMEM((_K * _L,), jnp.float32),
            pltpu.VMEM((128, 128), jnp.int32),
            pltpu.VMEM((_APW, 128), jnp.int32),
            pltpu.VMEM((_APW, 128), jnp.int32),
            pltpu.VMEM((128, 128), jnp.float32),
            pltpu.VMEM((_L,), jnp.float32),
        ],
    )
    return f(p0t.reshape(-1), p1.reshape(-1), p2.reshape(-1), cm0, cm1, cm2)


def _tc_product_body(p0t_ref, p1_ref, p2_ref, cm0_ref, cm1_ref, cm2_ref,
                     raw_ref, ps_ref):
    kiota_c = lax.broadcasted_iota(jnp.int32, (_K, 128), 0)
    kiota_b = lax.broadcasted_iota(jnp.int32, (128, _K), 1)
    cm0v = cm0_ref[...]
    m0 = [(cm0v == k).astype(jnp.float32) for k in range(_K)]
    acc = jnp.zeros((128,), jnp.float32)
    for a in range(8):
        oh1 = (cm1_ref[a][None, :] == kiota_c).astype(jnp.float32)
        t1 = lax.dot_general(p1_ref[...], oh1, (((0,), (0,)), ((), ())),
                             preferred_element_type=jnp.float32)
        oh2 = (cm2_ref[a][:, None] == kiota_b).astype(jnp.float32)
        t2 = lax.dot_general(oh2, p2_ref[...], (((1,), (0,)), ((), ())),
                             preferred_element_type=jnp.float32)
        t0 = m0[0] * p0t_ref[a, 0]
        for k in range(1, _K):
            t0 = t0 + m0[k] * p0t_ref[a, k]
        slab = t0 * t1 * t2
        raw_ref[a] = slab
        acc = acc + jnp.sum(slab, axis=0)
    ps_ref[0, 0] = acc


def _compute_product_tc(p0t, p1, p2, cm0, cm1, cm2):
    return pl.pallas_call(
        _tc_product_body,
        grid=(_ASPLIT // 8,),
        in_specs=[
            pl.BlockSpec((8, _K), lambda g: (g, 0)),
            pl.BlockSpec((_K, 128), lambda g: (0, 0)),
            pl.BlockSpec((_K, 128), lambda g: (0, 0)),
            pl.BlockSpec((128, 128), lambda g: (0, 0)),
            pl.BlockSpec((8, 128), lambda g: (g, 0)),
            pl.BlockSpec((8, 128), lambda g: (g, 0)),
        ],
        out_specs=[
            pl.BlockSpec((8, 128, 128), lambda g: (g, 0, 0)),
            pl.BlockSpec((1, 1, 128), lambda g: (g, 0, 0)),
        ],
        out_shape=[
            jax.ShapeDtypeStruct((_ASPLIT, 128, 128), jnp.float32),
            jax.ShapeDtypeStruct((_ASPLIT // 8, 1, 128), jnp.float32),
        ],
    )(p0t, p1, p2, cm0, cm1, cm2)


_NT = _ASPLIT // 8


def _scale_body(rawtc_ref, rawsc_ref, pstc_ref, pssc_ref, out_ref):
    total = jnp.sum(pstc_ref[...]) + jnp.sum(pssc_ref[...])
    g = pl.program_id(0)

    @pl.when(g < _NT)
    def _():
        out_ref[...] = jnp.where(total > 0, rawtc_ref[...] / total,
                                 1.0 / _JOINT)

    @pl.when(g >= _NT)
    def _():
        out_ref[...] = jnp.where(total > 0, rawsc_ref[...] / total,
                                 1.0 / _JOINT)


def _scale(raw_tc, raw_sc, ps_tc, ps_sc):
    return pl.pallas_call(
        _scale_body,
        grid=(32,),
        in_specs=[
            pl.BlockSpec((8, 128, 128),
                         lambda g: (jnp.minimum(g, _NT - 1), 0, 0)),
            pl.BlockSpec((8, 128, 128),
                         lambda g: (jnp.maximum(g - _NT, 0), 0, 0)),
            pl.BlockSpec((_NT, 1, 128), lambda g: (0, 0, 0)),
            pl.BlockSpec((_NW, _L), lambda g: (0, 0)),
        ],
        out_specs=pl.BlockSpec((8, 128, 128), lambda g: (g, 0, 0)),
        out_shape=jax.ShapeDtypeStruct((256, 128, 128), jnp.float32),
    )(raw_tc, raw_sc, ps_tc, ps_sc)


@jax.jit
def kernel(states_0, states_1, states_2, tm_0, tm_1, tm_2,
           norm_0, norm_1, norm_2, cm_0, cm_1, cm_2):
    states = (states_0, states_1, states_2)
    tms = (tm_0, tm_1, tm_2)
    ws = (jnp.ones((_K, _S), jnp.float32), norm_1,
          jnp.ones((_K, _S), jnp.float32))
    tm_flat = [tms[i].reshape(_K, _VOCAB[i], _S * _S) for i in range(3)]
    w_flat = [(states[i][None, :, None] * ws[i][:, None, :])
              .reshape(_K, 1, _S * _S) for i in range(3)]

    p0, p1, p2 = _compute_tables(*tm_flat, *w_flat)

    cm0 = cm_0.reshape(128, 128)
    cm1 = cm_1.reshape(256, 128)
    cm2 = cm_2.reshape(256, 128)
    p0t = p0.T
    raw_sc, ps_sc = _compute_product(p0t[_ASPLIT:], p1, p2, cm0,
                                     cm1[_ASPLIT:], cm2[_ASPLIT:])
    raw_tc, ps_tc = _compute_product_tc(p0t[:_ASPLIT], p1, p2, cm0,
                                        cm1[:_ASPLIT], cm2[:_ASPLIT])
    out = _scale(raw_tc, raw_sc, ps_tc, ps_sc)
    return out.reshape(-1)


# speedup vs baseline: 1.3436x; 1.3436x over previous
import functools

import jax
import jax.numpy as jnp
from jax import lax
from jax.experimental import pallas as pl
from jax.experimental.pallas import tpu as pltpu
from jax.experimental.pallas import tpu_sc as plsc

_VOCAB = (256, 128, 128)
_K = 16
_S = 32
_JOINT = 256 * 128 * 128

_NC = 2
_NS = 16
_NW = _NC * _NS
_L = 16

_ASPLIT = 224
_ASC = 256 - _ASPLIT
_APW = _ASC // _NW


def _tables_body(tm0_ref, tm1_ref, tm2_ref, w0_ref, w1_ref, w2_ref,
                 p0_ref, p1_ref, p2_ref):
    for tm_ref, w_ref, p_ref in ((tm0_ref, w0_ref, p0_ref),
                                 (tm1_ref, w1_ref, p1_ref),
                                 (tm2_ref, w2_ref, p2_ref)):
        x = tm_ref[0]
        w = w_ref[0]
        vals = jnp.sum(x * w, axis=1)
        vals = jnp.abs(vals) + 1e-9
        p_ref[0, 0] = vals / jnp.sum(vals)


def _compute_tables(tm0, tm1, tm2, w0, w1, w2):
    out = pl.pallas_call(
        _tables_body,
        grid=(_K,),
        in_specs=[
            pl.BlockSpec((1, _VOCAB[0], _S * _S), lambda k: (k, 0, 0)),
            pl.BlockSpec((1, _VOCAB[1], _S * _S), lambda k: (k, 0, 0)),
            pl.BlockSpec((1, _VOCAB[2], _S * _S), lambda k: (k, 0, 0)),
            pl.BlockSpec((1, 1, _S * _S), lambda k: (k, 0, 0)),
            pl.BlockSpec((1, 1, _S * _S), lambda k: (k, 0, 0)),
            pl.BlockSpec((1, 1, _S * _S), lambda k: (k, 0, 0)),
        ],
        out_specs=[
            pl.BlockSpec((1, 1, _VOCAB[0]), lambda k: (k, 0, 0)),
            pl.BlockSpec((1, 1, _VOCAB[1]), lambda k: (k, 0, 0)),
            pl.BlockSpec((1, 1, _VOCAB[2]), lambda k: (k, 0, 0)),
        ],
        out_shape=[
            jax.ShapeDtypeStruct((_K, 1, _VOCAB[0]), jnp.float32),
            jax.ShapeDtypeStruct((_K, 1, _VOCAB[1]), jnp.float32),
            jax.ShapeDtypeStruct((_K, 1, _VOCAB[2]), jnp.float32),
        ],
    )(tm0, tm1, tm2, w0, w1, w2)
    return [o.reshape(_K, v) for o, v in zip(out, _VOCAB)]


_P1STRIDE = _K * _VOCAB[1] + 1


def _sc_product_body(p0t_h, p1_h, p2_h, cm0_h, cm1_h, cm2_h,
                     raw_h, psum_h,
                     p0t_v, p1_v, p2_v, p1rep_v, q0_v,
                     cm0_v, cm1_v, cm2_v, ob, sv):
    cid = lax.axis_index("c")
    sid = lax.axis_index("s")
    wid = sid * _NC + cid
    a0 = wid * _APW
    lane = lax.iota(jnp.int32, _L)
    pltpu.sync_copy(p0t_h.at[pl.ds(a0 * _K, _APW * _K)], p0t_v)
    pltpu.sync_copy(p1_h, p1_v)
    pltpu.sync_copy(p2_h, p2_v)
    pltpu.sync_copy(cm0_h, cm0_v)
    pltpu.sync_copy(cm1_h.at[pl.ds(a0, _APW)], cm1_v)
    pltpu.sync_copy(cm2_h.at[pl.ds(a0, _APW)], cm2_v)

    def pre0_body(i, _):
        for cc in range(128 // _L):
            v = cm0_v[i, pl.ds(cc * _L, _L)]
            cm0_v[i, pl.ds(cc * _L, _L)] = v * _L + lane
        return 0

    lax.fori_loop(0, 128, pre0_body, 0)

    lbases = [jnp.full((_L,), l * _P1STRIDE, jnp.int32) + lane
              for l in range(_L)]

    def rep_body(i, _):
        v = p1_v[pl.ds(i * _L, _L)]
        off = i * _L
        for l in range(_L):
            plsc.store_scatter(p1rep_v, [lbases[l] + off], v)
        return 0

    lax.fori_loop(0, (_K * _VOCAB[1]) // _L, rep_body, 0)
    lanestride = lane * _P1STRIDE

    def a_body(a, total):
        a_g = a0 + a
        q0 = p0t_v[pl.ds(a * _K, _K)]
        for k in range(_K):
            q0_v[pl.ds(k * _L, _L)] = jnp.full((_L,), q0[k], jnp.float32)
        idx1 = [cm1_v[a, pl.ds(cc * _L, _L)] * _VOCAB[1] + lanestride
                for cc in range(128 // _L)]

        @plsc.parallel_loop(0, 128 // _L, 1, unroll=2, carry=total)
        def bb_out(bb, tot, a=a, idx1=idx1):
            b0 = bb * _L
            r2v = cm2_v[a, pl.ds(b0, _L)]
            tot = list(tot)
            for j in range(_L):
                b = b0 + j
                r2 = r2v[j]
                b_vec = jnp.full((_L,), b, jnp.int32)
                for cc in range(128 // _L):
                    c0 = cc * _L
                    idx0 = cm0_v[b, pl.ds(c0, _L)]
                    t0 = plsc.load_gather(q0_v, [idx0])
                    t1 = plsc.load_gather(p1rep_v, [idx1[cc] + b_vec])
                    t2 = p2_v[pl.ds(r2 * _VOCAB[2] + c0, _L)]
                    prod = t0 * t1 * t2
                    ob[b, pl.ds(c0, _L)] = prod
                    tot[cc] = tot[cc] + prod
            return tuple(tot)

        total = bb_out
        pltpu.sync_copy(ob, raw_h.at[a_g])
        return total

    zeros = tuple(jnp.zeros((_L,), jnp.float32) for _ in range(128 // _L))
    total = lax.fori_loop(0, _APW, a_body, zeros)
    sv[...] = sum(total[1:], total[0])
    pltpu.sync_copy(sv, psum_h.at[wid])


def _compute_product(p0t, p1, p2, cm0, cm1, cm2):
    mesh = plsc.VectorSubcoreMesh(core_axis_name="c", subcore_axis_name="s",
                                  num_cores=_NC, num_subcores=_NS)
    f = pl.kernel(
        _sc_product_body,
        out_type=[
            jax.ShapeDtypeStruct((_ASC, 128, 128), jnp.float32),
            jax.ShapeDtypeStruct((_NW, _L), jnp.float32),
        ],
        mesh=mesh,
        compiler_params=pltpu.CompilerParams(use_tc_tiling_on_sc=False,
                                             needs_layout_passes=False),
        scratch_types=[
            pltpu.VMEM((_APW * _K,), jnp.float32),
            pltpu.VMEM((_K * _VOCAB[1],), jnp.float32),
            pltpu.VMEM((_K * _VOCAB[2],), jnp.float32),
            pltpu.VMEM((_L * _P1STRIDE,), jnp.float32),
            pltpu.VMEM((_K * _L,), jnp.float32),
            pltpu.VMEM((128, 128), jnp.int32),
            pltpu.VMEM((_APW, 128), jnp.int32),
            pltpu.VMEM((_APW, 128), jnp.int32),
            pltpu.VMEM((128, 128), jnp.float32),
            pltpu.VMEM((_L,), jnp.float32),
        ],
    )
    return f(p0t.reshape(-1), p1.reshape(-1), p2.reshape(-1), cm0, cm1, cm2)


def _tc_product_body(p0t_ref, p1_ref, p2_ref, cm0_ref, cm1_ref, cm2_ref,
                     raw_ref, ps_ref):
    kiota_c = lax.broadcasted_iota(jnp.int32, (_K, 128), 0)
    kiota_b = lax.broadcasted_iota(jnp.int32, (128, _K), 1)
    cm0v = cm0_ref[...]
    m0 = [(cm0v == k).astype(jnp.float32) for k in range(_K)]
    acc = jnp.zeros((128,), jnp.float32)
    for a in range(8):
        oh1 = (cm1_ref[a][None, :] == kiota_c).astype(jnp.float32)
        t1 = lax.dot_general(p1_ref[...], oh1, (((0,), (0,)), ((), ())),
                             preferred_element_type=jnp.float32)
        oh2 = (cm2_ref[a][:, None] == kiota_b).astype(jnp.float32)
        t2 = lax.dot_general(oh2, p2_ref[...], (((1,), (0,)), ((), ())),
                             preferred_element_type=jnp.float32)
        t0 = m0[0] * p0t_ref[a, 0]
        for k in range(1, _K):
            t0 = t0 + m0[k] * p0t_ref[a, k]
        slab = t0 * t1 * t2
        raw_ref[a] = slab
        acc = acc + jnp.sum(slab, axis=0)
    ps_ref[0, 0] = acc


def _compute_product_tc(p0t, p1, p2, cm0, cm1, cm2):
    return pl.pallas_call(
        _tc_product_body,
        grid=(_ASPLIT // 8,),
        in_specs=[
            pl.BlockSpec((8, _K), lambda g: (g, 0)),
            pl.BlockSpec((_K, 128), lambda g: (0, 0)),
            pl.BlockSpec((_K, 128), lambda g: (0, 0)),
            pl.BlockSpec((128, 128), lambda g: (0, 0)),
            pl.BlockSpec((8, 128), lambda g: (g, 0)),
            pl.BlockSpec((8, 128), lambda g: (g, 0)),
        ],
        out_specs=[
            pl.BlockSpec((8, 128, 128), lambda g: (g, 0, 0)),
            pl.BlockSpec((1, 1, 128), lambda g: (g, 0, 0)),
        ],
        out_shape=[
            jax.ShapeDtypeStruct((_ASPLIT, 128, 128), jnp.float32),
            jax.ShapeDtypeStruct((_ASPLIT // 8, 1, 128), jnp.float32),
        ],
    )(p0t, p1, p2, cm0, cm1, cm2)


_NT = _ASPLIT // 8


def _scale_body(rawtc_ref, rawsc_ref, pstc_ref, pssc_ref, out_ref):
    total = jnp.sum(pstc_ref[...]) + jnp.sum(pssc_ref[...])
    g = pl.program_id(0)

    @pl.when(g < _NT)
    def _():
        out_ref[...] = jnp.where(total > 0, rawtc_ref[...] / total,
                                 1.0 / _JOINT)

    @pl.when(g >= _NT)
    def _():
        out_ref[...] = jnp.where(total > 0, rawsc_ref[...] / total,
                                 1.0 / _JOINT)


def _scale(raw_tc, raw_sc, ps_tc, ps_sc):
    return pl.pallas_call(
        _scale_body,
        grid=(32,),
        in_specs=[
            pl.BlockSpec((8, 128, 128),
                         lambda g: (jnp.minimum(g, _NT - 1), 0, 0)),
            pl.BlockSpec((8, 128, 128),
                         lambda g: (jnp.maximum(g - _NT, 0), 0, 0)),
            pl.BlockSpec((_NT, 1, 128), lambda g: (0, 0, 0)),
            pl.BlockSpec((_NW, _L), lambda g: (0, 0)),
        ],
        out_specs=pl.BlockSpec((8, 128, 128), lambda g: (g, 0, 0)),
        out_shape=jax.ShapeDtypeStruct((256, 128, 128), jnp.float32),
    )(raw_tc, raw_sc, ps_tc, ps_sc)


@jax.jit
def kernel(states_0, states_1, states_2, tm_0, tm_1, tm_2,
           norm_0, norm_1, norm_2, cm_0, cm_1, cm_2):
    states = (states_0, states_1, states_2)
    tms = (tm_0, tm_1, tm_2)
    ws = (jnp.ones((_K, _S), jnp.float32), norm_1,
          jnp.ones((_K, _S), jnp.float32))
    tm_flat = [tms[i].reshape(_K, _VOCAB[i], _S * _S) for i in range(3)]
    w_flat = [(states[i][None, :, None] * ws[i][:, None, :])
              .reshape(_K, 1, _S * _S) for i in range(3)]

    p0, p1, p2 = _compute_tables(*tm_flat, *w_flat)

    cm0 = cm_0.reshape(128, 128)
    cm1 = cm_1.reshape(256, 128)
    cm2 = cm_2.reshape(256, 128)
    p0t = p0.T
    raw_sc, ps_sc = _compute_product(p0t[_ASPLIT:], p1, p2, cm0,
                                     cm1[_ASPLIT:], cm2[_ASPLIT:])
    raw_tc, ps_tc = _compute_product_tc(p0t[:_ASPLIT], p1, p2, cm0,
                                        cm1[:_ASPLIT], cm2[:_ASPLIT])
    out = _scale(raw_tc, raw_sc, ps_tc, ps_sc)
    return out.reshape(-1)
